# baseline (speedup 1.0000x reference)
import dataclasses
import functools

import jax
import jax.numpy as jnp
from jax import lax
from jax.experimental import pallas as pl
from jax.experimental.pallas import tpu as pltpu
from jax.experimental.pallas import tpu_sc as plsc

N = 10000
D = 128
DH = D // 2
NC = 2
NS = 16
LANES = 16
EB = 80
NBUF = 4
CH = 50

_SC_PARAMS = pltpu.CompilerParams(use_tc_tiling_on_sc=False)
if "needs_layout_passes" in pltpu.CompilerParams.__dataclass_fields__:
    _SC_PARAMS = dataclasses.replace(_SC_PARAMS, needs_layout_passes=False)

NPAD = 10016
ROWS_PER_TILE = NPAD // NS


def _sc_message_layer(xh, edges):
    nb = edges.shape[0] // NS
    assert nb % CH == 0 and nb % 4 == 2 and nb >= 14
    mesh = plsc.VectorSubcoreMesh(core_axis_name="c", subcore_axis_name="s")

    @functools.partial(
        pl.kernel,
        out_type=jax.ShapeDtypeStruct((NC, NPAD, DH), jnp.float32),
        mesh=mesh,
        compiler_params=_SC_PARAMS,
        scratch_types=[
            pltpu.VMEM((2, CH, 3, EB), jnp.int32),
            pltpu.VMEM((NBUF, EB, DH), jnp.float32),
            pltpu.VMEM_SHARED((NPAD, DH), jnp.float32),
            pltpu.SemaphoreType.DMA((NBUF,)),
            pltpu.SemaphoreType.DMA((NBUF,)),
        ],
    )
    def sc_kernel(x_hbm, e_hbm, out_hbm, st_v, rows_v, aggr_sh, gsem, ssem):
        cid = lax.axis_index("c")
        sid = lax.axis_index("s")
        x_view = x_hbm.at[cid]
        tile_row0 = sid * nb

        zvec = jnp.zeros((LANES,), jnp.float32)

        @pl.loop(0, EB)
        def _(i):
            for b in range(NBUF):
                for c in range(DH // LANES):
                    rows_v[b, i, pl.ds(c * LANES, LANES)] = zvec

        zb = rows_v.at[0]
        out_base = sid * ROWS_PER_TILE

        @pl.loop(0, ROWS_PER_TILE // EB)
        def _(z):
            pltpu.sync_copy(zb, aggr_sh.at[pl.ds(out_base + z * EB, EB)])

        rem_rows = ROWS_PER_TILE % EB
        if rem_rows:
            pltpu.sync_copy(
                rows_v.at[0, pl.ds(0, rem_rows)],
                aggr_sh.at[pl.ds(out_base + ROWS_PER_TILE - rem_rows,
                                 rem_rows)])
        plsc.subcore_barrier()

        def src_ref(cb, row):
            return st_v.at[cb, row, 0]

        def dst_ref(cb, row):
            return st_v.at[cb, row, 1]

        def load_chunk(k0, cb):
            pltpu.sync_copy(e_hbm.at[pl.ds(tile_row0 + k0, CH)],
                            st_v.at[cb])

        def start_gather(cb, row, b):
            pltpu.async_copy(x_view.at[src_ref(cb, row)], rows_v.at[b],
                             gsem.at[b])

        def wait_gather(cb, row, b):
            pltpu.make_async_copy(x_view.at[src_ref(cb, row)], rows_v.at[b],
                                  gsem.at[b]).wait()

        def start_scatter(cb, row, b):
            pltpu.async_copy(rows_v.at[b], aggr_sh.at[dst_ref(cb, row)],
                             ssem.at[b], add=True)

        def wait_scatter(cb, row, b):
            pltpu.make_async_copy(rows_v.at[b], aggr_sh.at[dst_ref(cb, row)],
                                  ssem.at[b]).wait()

        def compute(cb, row, b):
            wrow = st_v.at[cb, row, 2]

            @pl.loop(0, EB, unroll=16)
            def _(i):
                ii = jnp.full((LANES,), i, dtype=jnp.int32)
                wb = plsc.bitcast(plsc.load_gather(wrow, [ii]), jnp.float32)
                for c in range(DH // LANES):
                    sl = (b, i, pl.ds(c * LANES, LANES))
                    rows_v[sl] = jnp.maximum(rows_v[sl] + wb, 0.0)

        i0 = jnp.int32(0)
        load_chunk(i0, i0)
        start_gather(i0, i0, 0)
        start_gather(i0, jnp.int32(1), 1)

        def step(k, b, drain, prefetch):
            row = lax.rem(k, CH)
            cb = lax.rem(lax.div(k, CH), 2)
            if drain:
                wait_scatter(cb, row, (b + 2) % NBUF)
            if prefetch:
                row2 = lax.rem(k + 2, CH)
                cb2 = lax.rem(lax.div(k + 2, CH), 2)

                @pl.when(row2 == 0)
                def _():
                    load_chunk(k + 2, cb2)

                start_gather(cb2, row2, (b + 2) % NBUF)
            wait_gather(cb, row, b)
            compute(cb, row, b)
            start_scatter(cb, row, b)

        step(i0, 0, False, True)
        step(jnp.int32(1), 1, False, True)
        step(jnp.int32(2), 2, True, True)
        step(jnp.int32(3), 3, True, True)

        @pl.loop(4, nb - 2, step=4)
        def _(k0):
            for j in range(4):
                step(k0 + j, j, True, True)

        step(jnp.int32(nb - 2), (nb - 2) % NBUF, True, False)
        step(jnp.int32(nb - 1), (nb - 1) % NBUF, True, False)
        wait_scatter(i0, i0, (nb - 2) % NBUF)
        wait_scatter(i0, i0, (nb - 1) % NBUF)

        plsc.subcore_barrier()
        pltpu.sync_copy(aggr_sh.at[pl.ds(out_base, ROWS_PER_TILE)],
                        out_hbm.at[cid, pl.ds(out_base, ROWS_PER_TILE)])

    return sc_kernel(xh, edges)


def _tc_dense_layer(xh, parts, W, b2d, act):
    m_blk = 2000
    c = W.shape[1]

    def body(x_ref, p_ref, w_ref, b_ref, o_ref):
        s = jnp.concatenate(
            [x_ref[0] + p_ref[0], x_ref[1] + p_ref[1]], axis=-1)
        acc = lax.dot_general(s, w_ref[...], (((1,), (0,)), ((), ())),
                              preferred_element_type=jnp.float32,
                              precision=lax.Precision.HIGHEST)
        acc = acc + b_ref[...]
        if act == "relu":
            o_ref[0] = jnp.maximum(acc[:, :DH], 0.0)
            o_ref[1] = jnp.maximum(acc[:, DH:], 0.0)
        else:
            acc = acc - jnp.max(acc, axis=-1, keepdims=True)
            acc = jnp.exp(acc)
            o_ref[...] = acc / jnp.sum(acc, axis=-1, keepdims=True)

    if act == "relu":
        out_shape = jax.ShapeDtypeStruct((2, N, DH), jnp.float32)
        out_spec = pl.BlockSpec((2, m_blk, DH), lambda i: (0, i, 0))
    else:
        out_shape = jax.ShapeDtypeStruct((N, c), jnp.float32)
        out_spec = pl.BlockSpec((m_blk, c), lambda i: (i, 0))

    return pl.pallas_call(
        body,
        grid=(N // m_blk,),
        in_specs=[
            pl.BlockSpec((2, m_blk, DH), lambda i: (0, i, 0)),
            pl.BlockSpec((2, m_blk, DH), lambda i: (0, i, 0)),
            pl.BlockSpec((D, c), lambda i: (0, 0)),
            pl.BlockSpec((1, c), lambda i: (0, 0)),
        ],
        out_specs=out_spec,
        out_shape=out_shape,
    )(xh, parts, W, b2d)


def kernel(features, edge_index, edge_weight, W0, b0, W1, b1, W2, b2):
    e = edge_index.shape[1]
    blk = NS * EB
    nb = -(-e // blk)
    nb += (-nb) % CH
    epad = nb * blk - e

    src = edge_index[0].astype(jnp.int32)
    dst = edge_index[1].astype(jnp.int32)
    wbits = lax.bitcast_convert_type(edge_weight.astype(jnp.float32),
                                     jnp.int32)
    if epad:
        src = jnp.concatenate([src, jnp.zeros((epad,), jnp.int32)])
        dst = jnp.concatenate([dst, jnp.full((epad,), N, jnp.int32)])
        wbits = jnp.concatenate([wbits, jnp.zeros((epad,), jnp.int32)])
    edges = jnp.stack(
        [src.reshape(NS * nb, EB), dst.reshape(NS * nb, EB),
         wbits.reshape(NS * nb, EB)], axis=1)

    xh = jnp.stack([features[:, :DH], features[:, DH:]])
    for W, b, act in ((W0, b0, "relu"), (W1, b1, "relu"), (W2, b2, "softmax")):
        parts = _sc_message_layer(xh, edges)
        xh = _tc_dense_layer(xh, parts, W, b.reshape(1, -1), act)
    return xh


# speedup vs baseline: 2.1518x; 2.1518x over previous
import dataclasses
import functools

import jax
import jax.numpy as jnp
from jax import lax
from jax.experimental import pallas as pl
from jax.experimental.pallas import tpu as pltpu
from jax.experimental.pallas import tpu_sc as plsc

N = 10000
D = 128
DH = D // 2
NC = 2
NS = 16
LANES = 16
EB = 80
NBUF = 4
CH = 50

_SC_PARAMS = pltpu.CompilerParams(use_tc_tiling_on_sc=False)
if "needs_layout_passes" in pltpu.CompilerParams.__dataclass_fields__:
    _SC_PARAMS = dataclasses.replace(_SC_PARAMS, needs_layout_passes=False)

NPAD = 10016
ROWS_PER_TILE = NPAD // NS


def _sc_message_layer(xh, edges):
    nb = edges.shape[0] // NS
    assert nb % CH == 0 and nb % 4 == 2 and nb >= 14
    mesh = plsc.VectorSubcoreMesh(core_axis_name="c", subcore_axis_name="s")

    @functools.partial(
        pl.kernel,
        out_type=jax.ShapeDtypeStruct((NC, NPAD, DH), jnp.float32),
        mesh=mesh,
        compiler_params=_SC_PARAMS,
        scratch_types=[
            pltpu.VMEM((2, CH, 3, EB), jnp.int32),
            pltpu.VMEM((NBUF, EB, DH), jnp.float32),
            pltpu.VMEM_SHARED((NPAD, DH), jnp.float32),
            pltpu.SemaphoreType.DMA((NBUF,)),
            pltpu.SemaphoreType.DMA((NBUF,)),
        ],
    )
    def sc_kernel(x_hbm, e_hbm, out_hbm, st_v, rows_v, aggr_sh, gsem, ssem):
        cid = lax.axis_index("c")
        sid = lax.axis_index("s")
        x_view = x_hbm.at[cid]
        tile_row0 = sid * nb

        zvec = jnp.zeros((LANES,), jnp.float32)

        @pl.loop(0, EB)
        def _(i):
            for b in range(NBUF):
                for c in range(DH // LANES):
                    rows_v[b, i, pl.ds(c * LANES, LANES)] = zvec

        zb = rows_v.at[0]
        out_base = sid * ROWS_PER_TILE

        @pl.loop(0, ROWS_PER_TILE // EB)
        def _(z):
            pltpu.sync_copy(zb, aggr_sh.at[pl.ds(out_base + z * EB, EB)])

        rem_rows = ROWS_PER_TILE % EB
        if rem_rows:
            pltpu.sync_copy(
                rows_v.at[0, pl.ds(0, rem_rows)],
                aggr_sh.at[pl.ds(out_base + ROWS_PER_TILE - rem_rows,
                                 rem_rows)])
        plsc.subcore_barrier()

        def src_ref(cb, row):
            return st_v.at[cb, row, 0]

        def dst_ref(cb, row):
            return st_v.at[cb, row, 1]

        def load_chunk(k0, cb):
            pltpu.sync_copy(e_hbm.at[pl.ds(tile_row0 + k0, CH)],
                            st_v.at[cb])

        def start_gather(cb, row, b):
            pltpu.async_copy(x_view.at[src_ref(cb, row)], rows_v.at[b],
                             gsem.at[b])

        def wait_gather(cb, row, b):
            pltpu.make_async_copy(x_view.at[src_ref(cb, row)], rows_v.at[b],
                                  gsem.at[b]).wait()

        def start_scatter(cb, row, b):
            pltpu.async_copy(rows_v.at[b], aggr_sh.at[dst_ref(cb, row)],
                             ssem.at[b], add=True)

        def wait_scatter(cb, row, b):
            pltpu.make_async_copy(rows_v.at[b], aggr_sh.at[dst_ref(cb, row)],
                                  ssem.at[b]).wait()

        def compute(cb, row, b):
            wrow = st_v.at[cb, row, 2]

            @pl.loop(0, EB, unroll=8)
            def _(i):
                ii = jnp.full((LANES,), i, dtype=jnp.int32)
                wb = plsc.bitcast(plsc.load_gather(wrow, [ii]), jnp.float32)
                for c in range(DH // LANES):
                    sl = (b, i, pl.ds(c * LANES, LANES))
                    rows_v[sl] = jnp.maximum(rows_v[sl] + wb, 0.0)

        i0 = jnp.int32(0)
        load_chunk(i0, i0)
        start_gather(i0, i0, 0)
        start_gather(i0, jnp.int32(1), 1)

        def step(k, b, drain, prefetch):
            row = lax.rem(k, CH)
            cb = lax.rem(lax.div(k, CH), 2)
            if drain:
                wait_scatter(cb, row, (b + 2) % NBUF)
            if prefetch:
                row2 = lax.rem(k + 2, CH)
                cb2 = lax.rem(lax.div(k + 2, CH), 2)

                @pl.when(row2 == 0)
                def _():
                    load_chunk(k + 2, cb2)

                start_gather(cb2, row2, (b + 2) % NBUF)
            wait_gather(cb, row, b)
            compute(cb, row, b)
            start_scatter(cb, row, b)

        step(i0, 0, False, True)
        step(jnp.int32(1), 1, False, True)
        step(jnp.int32(2), 2, True, True)
        step(jnp.int32(3), 3, True, True)

        @pl.loop(4, nb - 2, step=4)
        def _(k0):
            for j in range(4):
                step(k0 + j, j, True, True)

        step(jnp.int32(nb - 2), (nb - 2) % NBUF, True, False)
        step(jnp.int32(nb - 1), (nb - 1) % NBUF, True, False)
        wait_scatter(i0, i0, (nb - 2) % NBUF)
        wait_scatter(i0, i0, (nb - 1) % NBUF)

        plsc.subcore_barrier()
        pltpu.sync_copy(aggr_sh.at[pl.ds(out_base, ROWS_PER_TILE)],
                        out_hbm.at[cid, pl.ds(out_base, ROWS_PER_TILE)])

    return sc_kernel(xh, edges)


def _tc_dense_layer(xh, parts, W, b2d, act):
    m_blk = 2000
    c = W.shape[1]

    def body(x_ref, p_ref, w_ref, b_ref, o_ref):
        s = jnp.concatenate(
            [x_ref[0] + p_ref[0], x_ref[1] + p_ref[1]], axis=-1)
        acc = lax.dot_general(s, w_ref[...], (((1,), (0,)), ((), ())),
                              preferred_element_type=jnp.float32,
                              precision=lax.Precision.HIGHEST)
        acc = acc + b_ref[...]
        if act == "relu":
            o_ref[0] = jnp.maximum(acc[:, :DH], 0.0)
            o_ref[1] = jnp.maximum(acc[:, DH:], 0.0)
        else:
            acc = acc - jnp.max(acc, axis=-1, keepdims=True)
            acc = jnp.exp(acc)
            o_ref[...] = acc / jnp.sum(acc, axis=-1, keepdims=True)

    if act == "relu":
        out_shape = jax.ShapeDtypeStruct((2, N, DH), jnp.float32)
        out_spec = pl.BlockSpec((2, m_blk, DH), lambda i: (0, i, 0))
    else:
        out_shape = jax.ShapeDtypeStruct((N, c), jnp.float32)
        out_spec = pl.BlockSpec((m_blk, c), lambda i: (i, 0))

    return pl.pallas_call(
        body,
        grid=(N // m_blk,),
        in_specs=[
            pl.BlockSpec((2, m_blk, DH), lambda i: (0, i, 0)),
            pl.BlockSpec((2, m_blk, DH), lambda i: (0, i, 0)),
            pl.BlockSpec((D, c), lambda i: (0, 0)),
            pl.BlockSpec((1, c), lambda i: (0, 0)),
        ],
        out_specs=out_spec,
        out_shape=out_shape,
    )(xh, parts, W, b2d)


def kernel(features, edge_index, edge_weight, W0, b0, W1, b1, W2, b2):
    e = edge_index.shape[1]
    blk = NS * EB
    nb = -(-e // blk)
    nb += (-nb) % CH
    epad = nb * blk - e

    src = edge_index[0].astype(jnp.int32)
    dst = edge_index[1].astype(jnp.int32)
    wbits = lax.bitcast_convert_type(edge_weight.astype(jnp.float32),
                                     jnp.int32)
    if epad:
        src = jnp.concatenate([src, jnp.zeros((epad,), jnp.int32)])
        dst = jnp.concatenate([dst, jnp.full((epad,), N, jnp.int32)])
        wbits = jnp.concatenate([wbits, jnp.zeros((epad,), jnp.int32)])
    edges = jnp.stack(
        [src.reshape(NS * nb, EB), dst.reshape(NS * nb, EB),
         wbits.reshape(NS * nb, EB)], axis=1)

    xh = jnp.stack([features[:, :DH], features[:, DH:]])
    for W, b, act in ((W0, b0, "relu"), (W1, b1, "relu"), (W2, b2, "softmax")):
        parts = _sc_message_layer(xh, edges)
        xh = _tc_dense_layer(xh, parts, W, b.reshape(1, -1), act)
    return xh
